# baseline (device time: 6985 ns/iter reference)
import jax
import jax.numpy as jnp
from jax import lax
from jax.experimental import pallas as pl
from jax.experimental.pallas import tpu as pltpu


def kernel(x):
    m, n = x.shape
    h = n // 2

    def body(x_ref, out_ref, comm_ref, send_sems, recv_sems):
        my_x = lax.axis_index("x")
        my_y = lax.axis_index("y")
        partner = (1 - my_x, my_y)

        barrier_sem = pltpu.get_barrier_semaphore()
        pl.semaphore_signal(barrier_sem, inc=1)
        pl.semaphore_wait(barrier_sem, 1)

        def exchange(half):
            return pltpu.make_async_remote_copy(
                src_ref=comm_ref.at[0, :, pl.ds(half * h, h)],
                dst_ref=comm_ref.at[1, :, pl.ds(half * h, h)],
                send_sem=send_sems.at[half],
                recv_sem=recv_sems.at[half],
                device_id=partner,
                device_id_type=pl.DeviceIdType.MESH,
            )

        comm_ref[0, :, :h] = jnp.max(x_ref[:, :h], axis=0, keepdims=True)
        rdma0 = exchange(0)
        rdma0.start()

        comm_ref[0, :, h:] = jnp.max(x_ref[:, h:], axis=0, keepdims=True)
        rdma1 = exchange(1)
        rdma1.start()

        rdma0.wait()
        out_ref[:, :h] = jnp.maximum(comm_ref[0, :, :h], comm_ref[1, :, :h])
        rdma1.wait()
        out_ref[:, h:] = jnp.maximum(comm_ref[0, :, h:], comm_ref[1, :, h:])

    return pl.pallas_call(
        body,
        out_shape=jax.ShapeDtypeStruct((1, n), x.dtype),
        in_specs=[pl.BlockSpec(memory_space=pltpu.VMEM)],
        out_specs=pl.BlockSpec(memory_space=pltpu.VMEM),
        scratch_shapes=[
            pltpu.VMEM((2, 1, n), x.dtype),
            pltpu.SemaphoreType.DMA((2,)),
            pltpu.SemaphoreType.DMA((2,)),
        ],
        compiler_params=pltpu.CompilerParams(collective_id=0),
    )(x)


# device time: 6810 ns/iter; 1.0257x vs baseline; 1.0257x over previous
import jax
import jax.numpy as jnp
from jax import lax
from jax.experimental import pallas as pl
from jax.experimental.pallas import tpu as pltpu


def kernel(x):
    m, n = x.shape

    def body(x_ref, out_ref, comm_ref, send_sem, recv_sem):
        my_x = lax.axis_index("x")
        my_y = lax.axis_index("y")
        partner = (1 - my_x, my_y)

        barrier_sem = pltpu.get_barrier_semaphore()
        pl.semaphore_signal(
            barrier_sem, inc=1,
            device_id=partner, device_id_type=pl.DeviceIdType.MESH,
        )

        comm_ref[0, :, :] = jnp.max(x_ref[:, :], axis=0, keepdims=True)

        pl.semaphore_wait(barrier_sem, 1)

        rdma = pltpu.make_async_remote_copy(
            src_ref=comm_ref.at[0],
            dst_ref=comm_ref.at[1],
            send_sem=send_sem,
            recv_sem=recv_sem,
            device_id=partner,
            device_id_type=pl.DeviceIdType.MESH,
        )
        rdma.start()
        rdma.wait()

        out_ref[:, :] = jnp.maximum(comm_ref[0, :, :], comm_ref[1, :, :])

    return pl.pallas_call(
        body,
        out_shape=jax.ShapeDtypeStruct((1, n), x.dtype),
        in_specs=[pl.BlockSpec(memory_space=pltpu.VMEM)],
        out_specs=pl.BlockSpec(memory_space=pltpu.VMEM),
        scratch_shapes=[
            pltpu.VMEM((2, 1, n), x.dtype),
            pltpu.SemaphoreType.DMA,
            pltpu.SemaphoreType.DMA,
        ],
        compiler_params=pltpu.CompilerParams(collective_id=0),
    )(x)
